# Initial kernel scaffold; baseline (speedup 1.0000x reference)
import jax
import jax.numpy as jnp
from jax.experimental import pallas as pl


def kernel(indicators, neighborhood_ids, channels, popcounts):
    raise NotImplementedError("write your pallas kernel here")


# speedup vs baseline: 2.4034x; 2.4034x over previous
import functools

import jax
import jax.numpy as jnp
from jax import lax
from jax.experimental import pallas as pl
from jax.experimental.pallas import tpu as pltpu
from jax.experimental.pallas import tpu_sc as plsc

N_CHANNELS = 384
N_NEIGHB = 64
N_SPIKES = 1_000_000
N_QUERY_CH = 96
MIN_COVERAGE = 0.9

L = 16
NC = 2
NS = 16
NW = NC * NS
NV = N_SPIKES // L
BASE_V = NV // NW
REM_V = NV % NW
CHUNK_V = 512
N_CHUNKS = -(-(BASE_V + 1) // CHUNK_V)


def _sc_body(ind_hbm, ids_hbm, ch_hbm, pc_hbm,
             cov_hbm, cvd_hbm, nsp_hbm, out_hbm,
             ind_v, ch_v, pc_v, small_v, table_v, idbuf, outbuf, shared_tbl):
    cid = lax.axis_index("c")
    sid = lax.axis_index("s")
    iota = lax.iota(jnp.int32, L)

    @pl.when(sid == 0)
    def _phase_a():
        pltpu.sync_copy(ind_hbm, ind_v)
        pltpu.sync_copy(ch_hbm, ch_v)
        pltpu.sync_copy(pc_hbm, pc_v)
        nsp = jnp.int32(0)
        for jj in range(N_NEIGHB // L):
            def cnt_body(c, acc):
                return acc + ind_v[pl.ds(c * N_NEIGHB + jj * L, L)]
            cnt = lax.fori_loop(0, N_CHANNELS, cnt_body,
                                jnp.zeros((L,), jnp.float32))
            ssum = jnp.zeros((L,), jnp.float32)
            for g in range(N_QUERY_CH // L):
                for r in range(L):
                    chr_ = plsc.load_gather(ch_v, [g * L + ((iota + r) & (L - 1))])
                    ssum = ssum + plsc.load_gather(
                        ind_v, [chr_ * N_NEIGHB + jj * L + iota])
            cov = ssum / cnt
            cvd = cov >= MIN_COVERAGE
            masked = jnp.where(cvd, cov, jnp.float32(0.0))
            pc = pc_v[pl.ds(jj * L, L)]
            nsp = nsp + jnp.sum(jnp.where(cvd, pc, jnp.int32(0)))
            small_v[pl.ds(jj * L, L)] = cov
            small_v[pl.ds(N_NEIGHB + jj * L, L)] = masked
            table_v[pl.ds(jj * L, L)] = jnp.where(
                cvd, jnp.int32(1), jnp.int32(0)).astype(jnp.float32)
        small_v[pl.ds(2 * N_NEIGHB, L)] = jnp.full((L,), nsp, jnp.int32).astype(jnp.float32)
        pltpu.sync_copy(small_v.at[pl.ds(N_NEIGHB, N_NEIGHB)], shared_tbl)

        @pl.when(cid == 0)
        def _write_small():
            pltpu.sync_copy(small_v.at[pl.ds(0, N_NEIGHB)], cov_hbm)
            pltpu.sync_copy(table_v, cvd_hbm)
            pltpu.sync_copy(small_v.at[pl.ds(2 * N_NEIGHB, L)], nsp_hbm)

    plsc.subcore_barrier()
    pltpu.sync_copy(shared_tbl, table_v.at[pl.ds(0, N_NEIGHB)])

    w = sid * NC + cid
    n_w = BASE_V + jnp.where(w < REM_V, 1, 0)
    s_w = BASE_V * w + jnp.minimum(w, REM_V)
    for i in range(N_CHUNKS):
        coff = jnp.minimum(jnp.int32(i * CHUNK_V), n_w - CHUNK_V)
        base = (s_w + coff) * L
        pltpu.sync_copy(ids_hbm.at[pl.ds(base, CHUNK_V * L)], idbuf)

        def g_body(k, _):
            idx = idbuf[pl.ds(k * L, L)]
            outbuf[pl.ds(k * L, L)] = plsc.load_gather(
                table_v.at[pl.ds(0, N_NEIGHB)], [idx])
            return _
        lax.fori_loop(0, CHUNK_V, g_body, jnp.int32(0))
        pltpu.sync_copy(outbuf, out_hbm.at[pl.ds(base, CHUNK_V * L)])


@jax.jit
def _run(indf, ids, ch, pc):
    mesh = plsc.VectorSubcoreMesh(core_axis_name="c", subcore_axis_name="s",
                                  num_cores=NC, num_subcores=NS)
    f = pl.kernel(
        _sc_body,
        out_type=(
            jax.ShapeDtypeStruct((N_NEIGHB,), jnp.float32),
            jax.ShapeDtypeStruct((N_NEIGHB,), jnp.float32),
            jax.ShapeDtypeStruct((L,), jnp.float32),
            jax.ShapeDtypeStruct((N_SPIKES,), jnp.float32),
        ),
        mesh=mesh,
        compiler_params=pltpu.CompilerParams(needs_layout_passes=False),
        scratch_types=(
            pltpu.VMEM((N_CHANNELS * N_NEIGHB,), jnp.float32),
            pltpu.VMEM((N_QUERY_CH,), jnp.int32),
            pltpu.VMEM((N_NEIGHB,), jnp.int32),
            pltpu.VMEM((2 * N_NEIGHB + L,), jnp.float32),
            pltpu.VMEM((N_NEIGHB,), jnp.float32),
            pltpu.VMEM((CHUNK_V * L,), jnp.int32),
            pltpu.VMEM((CHUNK_V * L,), jnp.float32),
            pltpu.VMEM_SHARED((N_NEIGHB,), jnp.float32),
        ),
    )
    return f(indf, ids, ch, pc)


def kernel(indicators, neighborhood_ids, channels, popcounts):
    indf = indicators.reshape(-1).astype(jnp.float32)
    ids = neighborhood_ids.astype(jnp.int32)
    ch = channels.astype(jnp.int32)
    pc = popcounts.astype(jnp.int32)
    cov, cvd, nsp, spike_cov = _run(indf, ids, ch, pc)
    covered = cvd != 0.0
    n_spikes_covered = nsp[0].astype(jnp.int32)
    return cov, covered, n_spikes_covered, spike_cov


# speedup vs baseline: 3.2341x; 1.3456x over previous
import jax
import jax.numpy as jnp
from jax import lax
from jax.experimental import pallas as pl
from jax.experimental.pallas import tpu as pltpu
from jax.experimental.pallas import tpu_sc as plsc

N_CHANNELS = 384
N_NEIGHB = 64
N_SPIKES = 1_000_000
N_QUERY_CH = 96
MIN_COVERAGE = 0.9

L = 16
NC = 2
NS = 16
NW = NC * NS
NV = N_SPIKES // L
BASE_V = NV // NW
REM_V = NV % NW
CHUNK_V = 512
CW = CHUNK_V * L
N_CHUNKS = -(-(BASE_V + 1) // CHUNK_V)
U = 8


def _sc_body(ind_hbm, ids_hbm, ch_hbm, pc_hbm,
             cov_hbm, cvd_hbm, nsp_hbm, out_hbm,
             ind_v, ch_v, pc_v, small_v, table_v,
             id0, id1, ob0, ob1, shared_tbl,
             sin0, sin1, sout0, sout1):
    cid = lax.axis_index("c")
    sid = lax.axis_index("s")
    iota = lax.iota(jnp.int32, L)

    w = sid * NC + cid
    n_w = BASE_V + jnp.where(w < REM_V, 1, 0)
    s_w = BASE_V * w + jnp.minimum(w, REM_V)

    idbufs = [id0, id1]
    obufs = [ob0, ob1]
    sins = [sin0, sin1]
    souts = [sout0, sout1]

    def chunk_base(i):
        coff = jnp.minimum(jnp.int32(i * CHUNK_V), n_w - CHUNK_V)
        return (s_w + coff) * L

    in_d = [None] * N_CHUNKS
    in_d[0] = pltpu.async_copy(ids_hbm.at[pl.ds(chunk_base(0), CW)], id0, sin0)
    in_d[1] = pltpu.async_copy(ids_hbm.at[pl.ds(chunk_base(1), CW)], id1, sin1)

    @pl.when(sid == 0)
    def _phase_a():
        pltpu.sync_copy(ind_hbm, ind_v)
        pltpu.sync_copy(ch_hbm, ch_v)
        pltpu.sync_copy(pc_hbm, pc_v)
        zero = jnp.zeros((L,), jnp.float32)

        def cnt_body(c, accs):
            accs = list(accs)
            for u in range(4):
                ro = (c * 4 + u) * N_NEIGHB
                for jj in range(4):
                    accs[jj] = accs[jj] + ind_v[pl.ds(ro + jj * L, L)]
            return tuple(accs)
        cnts = lax.fori_loop(0, N_CHANNELS // 4, cnt_body, (zero,) * 4)

        ssums = [zero] * 4
        for g in range(N_QUERY_CH // L):
            for r in range(L):
                chr_ = plsc.load_gather(ch_v, [g * L + ((iota + r) & (L - 1))])
                ro = chr_ * N_NEIGHB
                for jj in range(4):
                    ssums[jj] = ssums[jj] + plsc.load_gather(
                        ind_v, [ro + jj * L + iota])

        nsp = jnp.int32(0)
        for jj in range(4):
            cov = ssums[jj] / cnts[jj]
            cvd = cov >= MIN_COVERAGE
            masked = jnp.where(cvd, cov, jnp.float32(0.0))
            pc = pc_v[pl.ds(jj * L, L)]
            nsp = nsp + jnp.sum(jnp.where(cvd, pc, jnp.int32(0)))
            small_v[pl.ds(jj * L, L)] = cov
            small_v[pl.ds(N_NEIGHB + jj * L, L)] = masked
            table_v[pl.ds(jj * L, L)] = jnp.where(
                cvd, jnp.float32(1.0), jnp.float32(0.0))
        small_v[pl.ds(2 * N_NEIGHB, L)] = jnp.full(
            (L,), nsp, jnp.int32).astype(jnp.float32)
        pltpu.sync_copy(small_v.at[pl.ds(N_NEIGHB, N_NEIGHB)], shared_tbl)

        @pl.when(cid == 0)
        def _write_small():
            pltpu.sync_copy(small_v.at[pl.ds(0, N_NEIGHB)], cov_hbm)
            pltpu.sync_copy(table_v, cvd_hbm)
            pltpu.sync_copy(small_v.at[pl.ds(2 * N_NEIGHB, L)], nsp_hbm)

    plsc.subcore_barrier()
    pltpu.sync_copy(shared_tbl, table_v)

    out_d = [None] * N_CHUNKS
    for i in range(N_CHUNKS):
        ib = idbufs[i % 2]
        ob = obufs[i % 2]
        if i >= 2:
            out_d[i - 2].wait()
        in_d[i].wait()

        def g_body(k, carry, ib=ib, ob=ob):
            for u in range(U):
                off = (k * U + u) * L
                idx = ib[pl.ds(off, L)]
                ob[pl.ds(off, L)] = plsc.load_gather(table_v, [idx])
            return carry
        lax.fori_loop(0, CHUNK_V // U, g_body, jnp.int32(0))

        if i + 2 < N_CHUNKS:
            in_d[i + 2] = pltpu.async_copy(
                ids_hbm.at[pl.ds(chunk_base(i + 2), CW)], ib, sins[i % 2])
        out_d[i] = pltpu.async_copy(
            ob, out_hbm.at[pl.ds(chunk_base(i), CW)], souts[i % 2])
    out_d[N_CHUNKS - 2].wait()
    out_d[N_CHUNKS - 1].wait()


@jax.jit
def _run(indf, ids, ch, pc):
    mesh = plsc.VectorSubcoreMesh(core_axis_name="c", subcore_axis_name="s",
                                  num_cores=NC, num_subcores=NS)
    f = pl.kernel(
        _sc_body,
        out_type=(
            jax.ShapeDtypeStruct((N_NEIGHB,), jnp.float32),
            jax.ShapeDtypeStruct((N_NEIGHB,), jnp.float32),
            jax.ShapeDtypeStruct((L,), jnp.float32),
            jax.ShapeDtypeStruct((N_SPIKES,), jnp.float32),
        ),
        mesh=mesh,
        compiler_params=pltpu.CompilerParams(needs_layout_passes=False),
        scratch_types=(
            pltpu.VMEM((N_CHANNELS * N_NEIGHB,), jnp.float32),
            pltpu.VMEM((N_QUERY_CH,), jnp.int32),
            pltpu.VMEM((N_NEIGHB,), jnp.int32),
            pltpu.VMEM((2 * N_NEIGHB + L,), jnp.float32),
            pltpu.VMEM((N_NEIGHB,), jnp.float32),
            pltpu.VMEM((CW,), jnp.int32),
            pltpu.VMEM((CW,), jnp.int32),
            pltpu.VMEM((CW,), jnp.float32),
            pltpu.VMEM((CW,), jnp.float32),
            pltpu.VMEM_SHARED((N_NEIGHB,), jnp.float32),
            pltpu.SemaphoreType.DMA,
            pltpu.SemaphoreType.DMA,
            pltpu.SemaphoreType.DMA,
            pltpu.SemaphoreType.DMA,
        ),
    )
    return f(indf, ids, ch, pc)


def kernel(indicators, neighborhood_ids, channels, popcounts):
    indf = indicators.reshape(-1).astype(jnp.float32)
    ids = neighborhood_ids.astype(jnp.int32)
    ch = channels.astype(jnp.int32)
    pc = popcounts.astype(jnp.int32)
    cov, cvd, nsp, spike_cov = _run(indf, ids, ch, pc)
    covered = cvd != 0.0
    n_spikes_covered = nsp[0].astype(jnp.int32)
    return cov, covered, n_spikes_covered, spike_cov


# speedup vs baseline: 3.3363x; 1.0316x over previous
import jax
import jax.numpy as jnp
from jax import lax
from jax.experimental import pallas as pl
from jax.experimental.pallas import tpu as pltpu
from jax.experimental.pallas import tpu_sc as plsc

N_CHANNELS = 384
N_NEIGHB = 64
N_SPIKES = 1_000_000
N_QUERY_CH = 96
MIN_COVERAGE = 0.9

L = 16
NC = 2
NS = 16
NW = NC * NS
NV = N_SPIKES // L
BASE_V = NV // NW
REM_V = NV % NW
CHUNK_V = 512
CW = CHUNK_V * L
N_CHUNKS = -(-(BASE_V + 1) // CHUNK_V)
U = 8
REP = N_NEIGHB * L


def _sc_body(ind_hbm, ids_hbm, ch_hbm, pc_hbm,
             cov_hbm, cvd_hbm, nsp_hbm, out_hbm,
             ind_v, ch_v, pc_v, small_v, rep_v,
             id0, id1, ob0, ob1, shared_rep,
             sin0, sin1, sout0, sout1):
    cid = lax.axis_index("c")
    sid = lax.axis_index("s")
    iota = lax.iota(jnp.int32, L)

    w = sid * NC + cid
    n_w = BASE_V + jnp.where(w < REM_V, 1, 0)
    s_w = BASE_V * w + jnp.minimum(w, REM_V)

    idbufs = [id0, id1]
    obufs = [ob0, ob1]
    sins = [sin0, sin1]
    souts = [sout0, sout1]

    def chunk_base(i):
        coff = jnp.minimum(jnp.int32(i * CHUNK_V), n_w - CHUNK_V)
        return (s_w + coff) * L

    in_d = [None] * N_CHUNKS
    in_d[0] = pltpu.async_copy(ids_hbm.at[pl.ds(chunk_base(0), CW)], id0, sin0)
    in_d[1] = pltpu.async_copy(ids_hbm.at[pl.ds(chunk_base(1), CW)], id1, sin1)

    @pl.when(sid == 0)
    def _phase_a():
        pltpu.sync_copy(ind_hbm, ind_v)
        pltpu.sync_copy(ch_hbm, ch_v)
        pltpu.sync_copy(pc_hbm, pc_v)
        zero = jnp.zeros((L,), jnp.float32)

        def cnt_body(c, accs):
            accs = list(accs)
            for u in range(4):
                for jj in range(4):
                    accs[jj] = accs[jj] + ind_v[c * 4 + u, pl.ds(jj * L, L)]
            return tuple(accs)
        cnts = lax.fori_loop(0, N_CHANNELS // 4, cnt_body, (zero,) * 4)

        def row_body(k, accs):
            g = k // L
            r = k % L
            chr_ = plsc.load_gather(ch_v, [g * L + ((iota + r) & (L - 1))])
            accs = list(accs)
            for jj in range(4):
                accs[jj] = accs[jj] + plsc.load_gather(
                    ind_v, [chr_, jj * L + iota])
            return tuple(accs)
        ssums = lax.fori_loop(0, N_QUERY_CH, row_body, (zero,) * 4)

        nsp = jnp.int32(0)
        for jj in range(4):
            cov = ssums[jj] / cnts[jj]
            cvd = cov >= MIN_COVERAGE
            masked = jnp.where(cvd, cov, jnp.float32(0.0))
            pc = pc_v[pl.ds(jj * L, L)]
            nsp = nsp + jnp.sum(jnp.where(cvd, pc, jnp.int32(0)))
            small_v[pl.ds(jj * L, L)] = cov
            small_v[pl.ds(N_NEIGHB + jj * L, L)] = jnp.where(
                cvd, jnp.float32(1.0), jnp.float32(0.0))
            jbase = (jj * L + iota) * L
            for c in range(L):
                plsc.store_scatter(rep_v, [jbase + c], masked)
        small_v[pl.ds(2 * N_NEIGHB, L)] = jnp.full(
            (L,), nsp, jnp.int32).astype(jnp.float32)
        pltpu.sync_copy(rep_v, shared_rep)

        @pl.when(cid == 0)
        def _write_small():
            pltpu.sync_copy(small_v.at[pl.ds(0, N_NEIGHB)], cov_hbm)
            pltpu.sync_copy(small_v.at[pl.ds(N_NEIGHB, N_NEIGHB)], cvd_hbm)
            pltpu.sync_copy(small_v.at[pl.ds(2 * N_NEIGHB, L)], nsp_hbm)

    plsc.subcore_barrier()
    pltpu.sync_copy(shared_rep, rep_v)

    out_d = [None] * N_CHUNKS
    for i in range(N_CHUNKS):
        ib = idbufs[i % 2]
        ob = obufs[i % 2]
        if i >= 2:
            out_d[i - 2].wait()
        in_d[i].wait()

        def g_body(k, carry, ib=ib, ob=ob):
            for u in range(U):
                off = (k * U + u) * L
                idx = ib[pl.ds(off, L)] * L + iota
                ob[pl.ds(off, L)] = plsc.load_gather(rep_v, [idx])
            return carry
        lax.fori_loop(0, CHUNK_V // U, g_body, jnp.int32(0))

        if i + 2 < N_CHUNKS:
            in_d[i + 2] = pltpu.async_copy(
                ids_hbm.at[pl.ds(chunk_base(i + 2), CW)], ib, sins[i % 2])
        out_d[i] = pltpu.async_copy(
            ob, out_hbm.at[pl.ds(chunk_base(i), CW)], souts[i % 2])
    out_d[N_CHUNKS - 2].wait()
    out_d[N_CHUNKS - 1].wait()


@jax.jit
def _run(ind, ids, ch, pc):
    mesh = plsc.VectorSubcoreMesh(core_axis_name="c", subcore_axis_name="s",
                                  num_cores=NC, num_subcores=NS)
    f = pl.kernel(
        _sc_body,
        out_type=(
            jax.ShapeDtypeStruct((N_NEIGHB,), jnp.float32),
            jax.ShapeDtypeStruct((N_NEIGHB,), jnp.float32),
            jax.ShapeDtypeStruct((L,), jnp.float32),
            jax.ShapeDtypeStruct((N_SPIKES,), jnp.float32),
        ),
        mesh=mesh,
        compiler_params=pltpu.CompilerParams(needs_layout_passes=False),
        scratch_types=(
            pltpu.VMEM((N_CHANNELS, N_NEIGHB), jnp.float32),
            pltpu.VMEM((N_QUERY_CH,), jnp.int32),
            pltpu.VMEM((N_NEIGHB,), jnp.int32),
            pltpu.VMEM((2 * N_NEIGHB + L,), jnp.float32),
            pltpu.VMEM((REP,), jnp.float32),
            pltpu.VMEM((CW,), jnp.int32),
            pltpu.VMEM((CW,), jnp.int32),
            pltpu.VMEM((CW,), jnp.float32),
            pltpu.VMEM((CW,), jnp.float32),
            pltpu.VMEM_SHARED((REP,), jnp.float32),
            pltpu.SemaphoreType.DMA,
            pltpu.SemaphoreType.DMA,
            pltpu.SemaphoreType.DMA,
            pltpu.SemaphoreType.DMA,
        ),
    )
    return f(ind, ids, ch, pc)


def kernel(indicators, neighborhood_ids, channels, popcounts):
    cov, cvd, nsp, spike_cov = _run(
        indicators.astype(jnp.float32), neighborhood_ids.astype(jnp.int32),
        channels.astype(jnp.int32), popcounts.astype(jnp.int32))
    covered = cvd != 0.0
    n_spikes_covered = nsp[0].astype(jnp.int32)
    return cov, covered, n_spikes_covered, spike_cov


# speedup vs baseline: 3.3388x; 1.0008x over previous
import jax
import jax.numpy as jnp
from jax import lax
from jax.experimental import pallas as pl
from jax.experimental.pallas import tpu as pltpu
from jax.experimental.pallas import tpu_sc as plsc

N_CHANNELS = 384
N_NEIGHB = 64
N_SPIKES = 1_000_000
N_QUERY_CH = 96
MIN_COVERAGE = 0.9

L = 16
NC = 2
NS = 16
NW = NC * NS
NV = N_SPIKES // L
BASE_V = NV // NW
REM_V = NV % NW
CHUNK_V = 512
CW = CHUNK_V * L
N_CHUNKS = -(-(BASE_V + 1) // CHUNK_V)
U = 8
REP = N_NEIGHB * L


def _sc_body(ind_hbm, ids_hbm, ch_hbm, pc_hbm,
             cov_hbm, cvd_hbm, nsp_hbm, out_hbm,
             ind_v, ch_v, pc_v, small_v, rep_v,
             id0, id1, ob0, ob1, shared_rep,
             sin0, sin1, sout0, sout1):
    cid = lax.axis_index("c")
    sid = lax.axis_index("s")
    iota = lax.iota(jnp.int32, L)

    w = sid * NC + cid
    n_w = BASE_V + jnp.where(w < REM_V, 1, 0)
    s_w = BASE_V * w + jnp.minimum(w, REM_V)

    idbufs = [id0, id1]
    obufs = [ob0, ob1]
    sins = [sin0, sin1]
    souts = [sout0, sout1]

    def chunk_base(i):
        coff = jnp.minimum(jnp.int32(i * CHUNK_V), n_w - CHUNK_V)
        return (s_w + coff) * L

    in_d = [None] * N_CHUNKS
    in_d[0] = pltpu.async_copy(ids_hbm.at[pl.ds(chunk_base(0), CW)], id0, sin0)
    in_d[1] = pltpu.async_copy(ids_hbm.at[pl.ds(chunk_base(1), CW)], id1, sin1)

    @pl.when(sid == 0)
    def _phase_a():
        pltpu.sync_copy(ind_hbm, ind_v)
        pltpu.sync_copy(ch_hbm, ch_v)
        pltpu.sync_copy(pc_hbm, pc_v)
        zero = jnp.zeros((L,), jnp.float32)

        def cnt_body(c, accs):
            accs = list(accs)
            for u in range(4):
                for jj in range(4):
                    accs[jj] = accs[jj] + ind_v[c * 4 + u, pl.ds(jj * L, L)]
            return tuple(accs)
        cnts = lax.fori_loop(0, N_CHANNELS // 4, cnt_body, (zero,) * 4)

        def row_body(k, accs):
            g = k // L
            r = k % L
            chr_ = plsc.load_gather(ch_v, [g * L + ((iota + r) & (L - 1))])
            accs = list(accs)
            for jj in range(4):
                accs[jj] = accs[jj] + plsc.load_gather(
                    ind_v, [chr_, jj * L + iota])
            return tuple(accs)
        ssums = lax.fori_loop(0, N_QUERY_CH, row_body, (zero,) * 4)

        nsp = jnp.int32(0)
        for jj in range(4):
            cov = ssums[jj] / cnts[jj]
            cvd = cov >= MIN_COVERAGE
            masked = jnp.where(cvd, cov, jnp.float32(0.0))
            pc = pc_v[pl.ds(jj * L, L)]
            nsp = nsp + jnp.sum(jnp.where(cvd, pc, jnp.int32(0)))
            small_v[pl.ds(jj * L, L)] = cov
            small_v[pl.ds(N_NEIGHB + jj * L, L)] = jnp.where(
                cvd, jnp.float32(1.0), jnp.float32(0.0))
            jbase = (jj * L + iota) * L
            for c in range(L):
                plsc.store_scatter(rep_v, [jbase + c], masked)
        small_v[pl.ds(2 * N_NEIGHB, L)] = jnp.full(
            (L,), nsp, jnp.int32).astype(jnp.float32)
        pltpu.sync_copy(rep_v, shared_rep)

        @pl.when(cid == 0)
        def _write_small():
            pltpu.sync_copy(small_v.at[pl.ds(0, N_NEIGHB)], cov_hbm)
            pltpu.sync_copy(small_v.at[pl.ds(N_NEIGHB, N_NEIGHB)], cvd_hbm)
            pltpu.sync_copy(small_v.at[pl.ds(2 * N_NEIGHB, L)], nsp_hbm)

    with jax.named_scope("barrier"):
        plsc.subcore_barrier()
        pltpu.sync_copy(shared_rep, rep_v)

    out_d = [None] * N_CHUNKS
    for i in range(N_CHUNKS):
        ib = idbufs[i % 2]
        ob = obufs[i % 2]
        with jax.named_scope(f"wait{i}"):
            if i >= 2:
                out_d[i - 2].wait()
            in_d[i].wait()

        def g_body(k, carry, ib=ib, ob=ob):
            for u in range(U):
                off = (k * U + u) * L
                idx = ib[pl.ds(off, L)] * L + iota
                ob[pl.ds(off, L)] = plsc.load_gather(rep_v, [idx])
            return carry
        with jax.named_scope(f"gather{i}"):
            lax.fori_loop(0, CHUNK_V // U, g_body, jnp.int32(0))

        if i + 2 < N_CHUNKS:
            in_d[i + 2] = pltpu.async_copy(
                ids_hbm.at[pl.ds(chunk_base(i + 2), CW)], ib, sins[i % 2])
        out_d[i] = pltpu.async_copy(
            ob, out_hbm.at[pl.ds(chunk_base(i), CW)], souts[i % 2])
    with jax.named_scope("drain"):
        out_d[N_CHUNKS - 2].wait()
        out_d[N_CHUNKS - 1].wait()


@jax.jit
def _run(ind, ids, ch, pc):
    mesh = plsc.VectorSubcoreMesh(core_axis_name="c", subcore_axis_name="s",
                                  num_cores=NC, num_subcores=NS)
    f = pl.kernel(
        _sc_body,
        out_type=(
            jax.ShapeDtypeStruct((N_NEIGHB,), jnp.float32),
            jax.ShapeDtypeStruct((N_NEIGHB,), jnp.float32),
            jax.ShapeDtypeStruct((L,), jnp.float32),
            jax.ShapeDtypeStruct((N_SPIKES,), jnp.float32),
        ),
        mesh=mesh,
        compiler_params=pltpu.CompilerParams(needs_layout_passes=False),
        scratch_types=(
            pltpu.VMEM((N_CHANNELS, N_NEIGHB), jnp.float32),
            pltpu.VMEM((N_QUERY_CH,), jnp.int32),
            pltpu.VMEM((N_NEIGHB,), jnp.int32),
            pltpu.VMEM((2 * N_NEIGHB + L,), jnp.float32),
            pltpu.VMEM((REP,), jnp.float32),
            pltpu.VMEM((CW,), jnp.int32),
            pltpu.VMEM((CW,), jnp.int32),
            pltpu.VMEM((CW,), jnp.float32),
            pltpu.VMEM((CW,), jnp.float32),
            pltpu.VMEM_SHARED((REP,), jnp.float32),
            pltpu.SemaphoreType.DMA,
            pltpu.SemaphoreType.DMA,
            pltpu.SemaphoreType.DMA,
            pltpu.SemaphoreType.DMA,
        ),
    )
    return f(ind, ids, ch, pc)


def kernel(indicators, neighborhood_ids, channels, popcounts):
    cov, cvd, nsp, spike_cov = _run(
        indicators.astype(jnp.float32), neighborhood_ids.astype(jnp.int32),
        channels.astype(jnp.int32), popcounts.astype(jnp.int32))
    covered = cvd != 0.0
    n_spikes_covered = nsp[0].astype(jnp.int32)
    return cov, covered, n_spikes_covered, spike_cov
